# baseline (speedup 1.0000x reference)
import functools

import jax
import jax.numpy as jnp
from jax import lax
from jax.experimental import pallas as pl
from jax.experimental.pallas import tpu as pltpu
from jax.experimental.pallas import tpu_sc as plsc


def _pool_body(x_ref, o_ref):
    o_ref[...] = jnp.mean(x_ref[...], axis=0, keepdims=True)


def _top2_body(m_ref, b_ref, idx_ref):
    logits = jnp.sum(m_ref[...], axis=1) + b_ref[0]
    iota = lax.iota(jnp.int32, logits.shape[0])
    i0 = jnp.argmax(logits).astype(jnp.int32)
    l2 = jnp.where(iota == i0, -jnp.inf, logits)
    i1 = jnp.argmax(l2).astype(jnp.int32)
    idx_ref[0] = i0
    idx_ref[1] = i1


def _route_sc(pooled, sel_WT):
    D = sel_WT.shape[1]
    NB = sel_WT.shape[0]
    mesh = plsc.VectorSubcoreMesh(core_axis_name="c", subcore_axis_name="s")

    @functools.partial(
        pl.kernel, mesh=mesh,
        out_type=jax.ShapeDtypeStruct((NB, 16), jnp.float32),
        scratch_types=[
            pltpu.VMEM((D,), jnp.float32),
            pltpu.VMEM((NB, D), jnp.float32),
            pltpu.VMEM((NB, 16), jnp.float32),
        ],
    )
    def route(pooled_hbm, wt_hbm, out_hbm, p_v, w_v, m_v):
        cid = lax.axis_index("c")
        sid = lax.axis_index("s")

        @pl.when(jnp.logical_and(cid == 0, sid == 0))
        def _():
            pltpu.sync_copy(pooled_hbm, p_v)
            pltpu.sync_copy(wt_hbm, w_v)
            for j in range(NB):
                acc = jnp.zeros((16,), jnp.float32)
                for c in range(D // 16):
                    lo, hi = c * 16, (c + 1) * 16
                    acc = acc + p_v[lo:hi] * w_v[j, lo:hi]
                m_v[j, :] = acc
            pltpu.sync_copy(m_v, out_hbm)

    return route(pooled, sel_WT)


def _fold(g_ref, beta_ref, w1_ref, b1_ref, w2_ref, b2_ref,
          w1g_ref, w2s_ref, b1p_ref, c2_ref, D):
    g_col = g_ref[0].reshape(D, 1)
    C = D // 4
    for c in range(4):
        lo, hi = c * C, (c + 1) * C
        w1g_ref[lo:hi, :] = (g_col[lo:hi, :]
                             * w1_ref[0, lo:hi, :]).astype(jnp.bfloat16)
        w2s_ref[lo:hi, :] = (0.3 * w2_ref[0, lo:hi, :]).astype(jnp.bfloat16)
    b1p_ref[...] = (jnp.dot(beta_ref[0], w1_ref[0],
                            preferred_element_type=jnp.float32) + b1_ref[0])
    c2_ref[...] = 0.3 * b2_ref[0]


def _apply_block(h, w1g_ref, w2s_ref, b1p_ref, c2_ref, tt):
    mu = jnp.mean(h, axis=1, keepdims=True)
    var = jnp.mean((h - mu) ** 2, axis=1, keepdims=True)
    z = ((h - mu) * jax.lax.rsqrt(var + 1e-5)).astype(jnp.bfloat16)
    a = jnp.tanh(jnp.dot(z, w1g_ref[...],
                         preferred_element_type=jnp.float32) + b1p_ref[...])
    q = jnp.dot(a.astype(jnp.bfloat16), w2s_ref[...],
                preferred_element_type=jnp.float32) + c2_ref[...]
    return h + q * tt


def _chain_body(idx_ref, h_ref,
                ga_ref, beta_a_ref, w1a_ref, b1a_ref, w2a_ref, b2a_ref,
                gb_ref, beta_b_ref, w1b_ref, b1b_ref, w2b_ref, b2b_ref,
                t_ref, o_ref,
                w1g_a, w2s_a, b1p_a, c2_a, w1g_b, w2s_b, b1p_b, c2_b):
    t = pl.program_id(0)
    D = h_ref.shape[1]

    @pl.when(t == 0)
    def _():
        _fold(ga_ref, beta_a_ref, w1a_ref, b1a_ref, w2a_ref, b2a_ref,
              w1g_a, w2s_a, b1p_a, c2_a, D)
        _fold(gb_ref, beta_b_ref, w1b_ref, b1b_ref, w2b_ref, b2b_ref,
              w1g_b, w2s_b, b1p_b, c2_b, D)

    tt = 1.0 + 0.1 * t_ref[0]
    h = h_ref[...]
    h = _apply_block(h, w1g_a, w2s_a, b1p_a, c2_a, tt)
    h = _apply_block(h, w1g_b, w2s_b, b1p_b, c2_b, tt)
    o_ref[...] = h


def kernel(embodied_input, disembodied_input, torsion_field, sel_W, sel_b,
           ln_g, ln_beta, W1, b1, W2, b2, max_active_blocks):
    B, S, D = embodied_input.shape
    NB = sel_b.shape[0]
    BS = B * S

    x0 = embodied_input[0]
    pooled = pl.pallas_call(
        _pool_body,
        in_specs=[pl.BlockSpec(memory_space=pltpu.VMEM)],
        out_specs=pl.BlockSpec(memory_space=pltpu.VMEM),
        out_shape=jax.ShapeDtypeStruct((1, D), jnp.float32),
    )(x0)
    partials = _route_sc(pooled.reshape(D), sel_W.T)
    top_idx = pl.pallas_call(
        _top2_body,
        in_specs=[
            pl.BlockSpec(memory_space=pltpu.VMEM),
            pl.BlockSpec(memory_space=pltpu.VMEM),
        ],
        out_specs=pl.BlockSpec(memory_space=pltpu.SMEM),
        out_shape=jax.ShapeDtypeStruct((2,), jnp.int32),
    )(partials, sel_b.reshape(1, NB))

    R = 1024
    T = BS // R
    S_per_batch = S

    def widx3(step):
        def f(t, s):
            del t
            return (s[step], 0, 0)
        return f

    def wspec(step, shape):
        return pl.BlockSpec(shape, widx3(step),
                            pipeline_mode=pl.Buffered(buffer_count=1))

    g3 = ln_g.reshape(NB, 1, D)
    be3 = ln_beta.reshape(NB, 1, D)
    b13 = b1.reshape(NB, 1, D)
    b23 = b2.reshape(NB, 1, D)

    h = pl.pallas_call(
        _chain_body,
        grid_spec=pltpu.PrefetchScalarGridSpec(
            num_scalar_prefetch=1,
            grid=(T,),
            in_specs=[
                pl.BlockSpec((R, D), lambda t, s: (t, 0)),
                wspec(0, (1, 1, D)), wspec(0, (1, 1, D)),
                wspec(0, (1, D, D)), wspec(0, (1, 1, D)),
                wspec(0, (1, D, D)), wspec(0, (1, 1, D)),
                wspec(1, (1, 1, D)), wspec(1, (1, 1, D)),
                wspec(1, (1, D, D)), wspec(1, (1, 1, D)),
                wspec(1, (1, D, D)), wspec(1, (1, 1, D)),
                pl.BlockSpec((1, 1, D),
                             lambda t, s: (t * R // S_per_batch, 0, 0)),
            ],
            out_specs=pl.BlockSpec((R, D), lambda t, s: (t, 0)),
            scratch_shapes=[
                pltpu.VMEM((D, D), jnp.bfloat16),
                pltpu.VMEM((D, D), jnp.bfloat16),
                pltpu.VMEM((1, D), jnp.float32),
                pltpu.VMEM((1, D), jnp.float32),
                pltpu.VMEM((D, D), jnp.bfloat16),
                pltpu.VMEM((D, D), jnp.bfloat16),
                pltpu.VMEM((1, D), jnp.float32),
                pltpu.VMEM((1, D), jnp.float32),
            ],
        ),
        out_shape=jax.ShapeDtypeStruct((BS, D), jnp.float32),
        compiler_params=pltpu.CompilerParams(
            dimension_semantics=("arbitrary",),
        ),
    )(top_idx, embodied_input.reshape(BS, D),
      g3, be3, W1, b13, W2, b23,
      g3, be3, W1, b13, W2, b23,
      torsion_field.reshape(B, 1, D))
    return h.reshape(B, S, D)


# speedup vs baseline: 1.0154x; 1.0154x over previous
import functools

import jax
import jax.numpy as jnp
from jax import lax
from jax.experimental import pallas as pl
from jax.experimental.pallas import tpu as pltpu
from jax.experimental.pallas import tpu_sc as plsc


def _pool_body(x_ref, o_ref):
    o_ref[...] = jnp.mean(x_ref[...], axis=0, keepdims=True)


def _top2_body(m_ref, b_ref, idx_ref):
    logits = jnp.sum(m_ref[...], axis=1) + b_ref[0]
    iota = lax.iota(jnp.int32, logits.shape[0])
    i0 = jnp.argmax(logits).astype(jnp.int32)
    l2 = jnp.where(iota == i0, -jnp.inf, logits)
    i1 = jnp.argmax(l2).astype(jnp.int32)
    idx_ref[0] = i0
    idx_ref[1] = i1


def _route_sc(pooled, sel_WT):
    D = sel_WT.shape[1]
    NB = sel_WT.shape[0]
    mesh = plsc.VectorSubcoreMesh(core_axis_name="c", subcore_axis_name="s")

    @functools.partial(
        pl.kernel, mesh=mesh,
        out_type=jax.ShapeDtypeStruct((NB, 16), jnp.float32),
        scratch_types=[
            pltpu.VMEM((D,), jnp.float32),
            pltpu.VMEM((D,), jnp.float32),
            pltpu.VMEM((16,), jnp.float32),
        ],
    )
    def route(pooled_hbm, wt_hbm, out_hbm, p_v, w_v, m_v):
        cid = lax.axis_index("c")
        sid = lax.axis_index("s")

        @pl.when(cid == 0)
        def _():
            pltpu.sync_copy(pooled_hbm, p_v)
            pltpu.sync_copy(wt_hbm.at[sid], w_v)
            acc = jnp.zeros((16,), jnp.float32)
            for c in range(D // 16):
                lo, hi = c * 16, (c + 1) * 16
                acc = acc + p_v[lo:hi] * w_v[lo:hi]
            m_v[...] = acc
            pltpu.sync_copy(m_v, out_hbm.at[sid])

    return route(pooled, sel_WT)


def _fold(g_ref, beta_ref, w1_ref, b1_ref, w2_ref, b2_ref,
          w1g_ref, w2s_ref, b1p_ref, c2_ref, D):
    g_col = g_ref[0].reshape(D, 1)
    C = D // 4
    for c in range(4):
        lo, hi = c * C, (c + 1) * C
        w1g_ref[lo:hi, :] = (g_col[lo:hi, :]
                             * w1_ref[0, lo:hi, :]).astype(jnp.bfloat16)
        w2s_ref[lo:hi, :] = (0.3 * w2_ref[0, lo:hi, :]).astype(jnp.bfloat16)
    b1p_ref[...] = (jnp.dot(beta_ref[0], w1_ref[0],
                            preferred_element_type=jnp.float32) + b1_ref[0])
    c2_ref[...] = 0.3 * b2_ref[0]


def _apply_block(h, w1g_ref, w2s_ref, b1p_ref, c2_ref, tt):
    mu = jnp.mean(h, axis=1, keepdims=True)
    var = jnp.mean((h - mu) ** 2, axis=1, keepdims=True)
    z = ((h - mu) * jax.lax.rsqrt(var + 1e-5)).astype(jnp.bfloat16)
    a = jnp.tanh(jnp.dot(z, w1g_ref[...],
                         preferred_element_type=jnp.float32) + b1p_ref[...])
    q = jnp.dot(a.astype(jnp.bfloat16), w2s_ref[...],
                preferred_element_type=jnp.float32) + c2_ref[...]
    return h + q * tt


def _chain_body(idx_ref, h_ref,
                ga_ref, beta_a_ref, w1a_ref, b1a_ref, w2a_ref, b2a_ref,
                gb_ref, beta_b_ref, w1b_ref, b1b_ref, w2b_ref, b2b_ref,
                t_ref, o_ref,
                w1g_a, w2s_a, b1p_a, c2_a, w1g_b, w2s_b, b1p_b, c2_b):
    t = pl.program_id(0)
    D = h_ref.shape[1]

    @pl.when(t == 0)
    def _():
        _fold(ga_ref, beta_a_ref, w1a_ref, b1a_ref, w2a_ref, b2a_ref,
              w1g_a, w2s_a, b1p_a, c2_a, D)
        _fold(gb_ref, beta_b_ref, w1b_ref, b1b_ref, w2b_ref, b2b_ref,
              w1g_b, w2s_b, b1p_b, c2_b, D)

    tt = 1.0 + 0.1 * t_ref[0]
    h = h_ref[...]
    h = _apply_block(h, w1g_a, w2s_a, b1p_a, c2_a, tt)
    h = _apply_block(h, w1g_b, w2s_b, b1p_b, c2_b, tt)
    o_ref[...] = h


def kernel(embodied_input, disembodied_input, torsion_field, sel_W, sel_b,
           ln_g, ln_beta, W1, b1, W2, b2, max_active_blocks):
    B, S, D = embodied_input.shape
    NB = sel_b.shape[0]
    BS = B * S

    x0 = embodied_input[0]
    pooled = pl.pallas_call(
        _pool_body,
        in_specs=[pl.BlockSpec(memory_space=pltpu.VMEM)],
        out_specs=pl.BlockSpec(memory_space=pltpu.VMEM),
        out_shape=jax.ShapeDtypeStruct((1, D), jnp.float32),
    )(x0)
    partials = _route_sc(pooled.reshape(D), sel_W.T)
    top_idx = pl.pallas_call(
        _top2_body,
        in_specs=[
            pl.BlockSpec(memory_space=pltpu.VMEM),
            pl.BlockSpec(memory_space=pltpu.VMEM),
        ],
        out_specs=pl.BlockSpec(memory_space=pltpu.SMEM),
        out_shape=jax.ShapeDtypeStruct((2,), jnp.int32),
    )(partials, sel_b.reshape(1, NB))

    R = 1024
    T = BS // R
    S_per_batch = S

    def widx3(step):
        def f(t, s):
            del t
            return (s[step], 0, 0)
        return f

    def wspec(step, shape):
        return pl.BlockSpec(shape, widx3(step),
                            pipeline_mode=pl.Buffered(buffer_count=1))

    g3 = ln_g.reshape(NB, 1, D)
    be3 = ln_beta.reshape(NB, 1, D)
    b13 = b1.reshape(NB, 1, D)
    b23 = b2.reshape(NB, 1, D)

    h = pl.pallas_call(
        _chain_body,
        grid_spec=pltpu.PrefetchScalarGridSpec(
            num_scalar_prefetch=1,
            grid=(T,),
            in_specs=[
                pl.BlockSpec((R, D), lambda t, s: (t, 0)),
                wspec(0, (1, 1, D)), wspec(0, (1, 1, D)),
                wspec(0, (1, D, D)), wspec(0, (1, 1, D)),
                wspec(0, (1, D, D)), wspec(0, (1, 1, D)),
                wspec(1, (1, 1, D)), wspec(1, (1, 1, D)),
                wspec(1, (1, D, D)), wspec(1, (1, 1, D)),
                wspec(1, (1, D, D)), wspec(1, (1, 1, D)),
                pl.BlockSpec((1, 1, D),
                             lambda t, s: (t * R // S_per_batch, 0, 0)),
            ],
            out_specs=pl.BlockSpec((R, D), lambda t, s: (t, 0)),
            scratch_shapes=[
                pltpu.VMEM((D, D), jnp.bfloat16),
                pltpu.VMEM((D, D), jnp.bfloat16),
                pltpu.VMEM((1, D), jnp.float32),
                pltpu.VMEM((1, D), jnp.float32),
                pltpu.VMEM((D, D), jnp.bfloat16),
                pltpu.VMEM((D, D), jnp.bfloat16),
                pltpu.VMEM((1, D), jnp.float32),
                pltpu.VMEM((1, D), jnp.float32),
            ],
        ),
        out_shape=jax.ShapeDtypeStruct((BS, D), jnp.float32),
        compiler_params=pltpu.CompilerParams(
            dimension_semantics=("arbitrary",),
        ),
    )(top_idx, embodied_input.reshape(BS, D),
      g3, be3, W1, b13, W2, b23,
      g3, be3, W1, b13, W2, b23,
      torsion_field.reshape(B, 1, D))
    return h.reshape(B, S, D)
